# baseline (speedup 1.0000x reference)
import functools

import jax
import jax.numpy as jnp
from jax import lax
from jax.experimental import pallas as pl
from jax.experimental.pallas import tpu as pltpu
from jax.experimental.pallas import tpu_sc as plsc

_B = 16384
_F = 32
_M = 5
_NC = 2
_NS = 16
_NW = _NC * _NS
_BPW = _B // _NW
_MPW = _BPW * _M
_MH = _MPW // 2
_LANES = 16


def _sc_gather(u_idx, i_idx, m_idx, ubh_idx, ibh_idx,
               user_table, item_table, meta_table, ub16, ib16):
    mesh = plsc.VectorSubcoreMesh(core_axis_name="c", subcore_axis_name="s")
    out_type = (
        jax.ShapeDtypeStruct((_B, _F), jnp.float32),
        jax.ShapeDtypeStruct((_B, _F), jnp.float32),
        jax.ShapeDtypeStruct((_B * _M, _F), jnp.float32),
        jax.ShapeDtypeStruct((_B, _LANES), jnp.float32),
        jax.ShapeDtypeStruct((_B, _LANES), jnp.float32),
    )
    scratch = [
        pltpu.VMEM((_BPW,), jnp.int32),
        pltpu.VMEM((_BPW,), jnp.int32),
        pltpu.VMEM((_MH,), jnp.int32),
        pltpu.VMEM((_BPW,), jnp.int32),
        pltpu.VMEM((_BPW,), jnp.int32),
        pltpu.VMEM((_BPW, _F), jnp.float32),
        pltpu.VMEM((_BPW, _F), jnp.float32),
        pltpu.VMEM((_MH, _F), jnp.float32),
        pltpu.VMEM((_BPW, _LANES), jnp.float32),
        pltpu.VMEM((_BPW, _LANES), jnp.float32),
        pltpu.SemaphoreType.DMA,
    ]

    @functools.partial(pl.kernel, mesh=mesh, out_type=out_type,
                       scratch_types=scratch,
                       compiler_params=pltpu.CompilerParams(
                           use_tc_tiling_on_sc=False))
    def k(u_idx_h, i_idx_h, m_idx_h, ub_idx_h, ib_idx_h,
          ut_h, it_h, mt_h, ubt_h, ibt_h,
          u_out, i_out, m_out, ub_out, ib_out,
          uiv, iiv, miv, ubiv, ibiv, urv, irv, mrv, ubrv, ibrv, sem):
        wid = lax.axis_index("s") * _NC + lax.axis_index("c")
        base = wid * _BPW

        pltpu.sync_copy(u_idx_h.at[pl.ds(base, _BPW)], uiv)
        pltpu.async_copy(ut_h.at[uiv], urv, sem).wait()
        pltpu.sync_copy(urv, u_out.at[pl.ds(base, _BPW)])

        pltpu.sync_copy(i_idx_h.at[pl.ds(base, _BPW)], iiv)
        pltpu.async_copy(it_h.at[iiv], irv, sem).wait()
        pltpu.sync_copy(irv, i_out.at[pl.ds(base, _BPW)])

        pltpu.sync_copy(ub_idx_h.at[pl.ds(base, _BPW)], ubiv)
        pltpu.async_copy(ubt_h.at[ubiv], ubrv, sem).wait()
        pltpu.sync_copy(ubrv, ub_out.at[pl.ds(base, _BPW)])

        pltpu.sync_copy(ib_idx_h.at[pl.ds(base, _BPW)], ibiv)
        pltpu.async_copy(ibt_h.at[ibiv], ibrv, sem).wait()
        pltpu.sync_copy(ibrv, ib_out.at[pl.ds(base, _BPW)])

        mbase = wid * _MPW
        for h in range(2):
            off = mbase + h * _MH
            pltpu.sync_copy(m_idx_h.at[pl.ds(off, _MH)], miv)
            pltpu.async_copy(mt_h.at[miv], mrv, sem).wait()
            pltpu.sync_copy(mrv, m_out.at[pl.ds(off, _MH)])

    return k(u_idx, i_idx, m_idx, ubh_idx, ibh_idx,
             user_table, item_table, meta_table, ub16, ib16)


_BBLK = 2048


def _tc_reduce(u_rows, i_rows, m_rows, ub_rows, ib_rows, u_lane, i_lane):
    def body(u_ref, it_ref, m_ref, ubr_ref, ibr_ref, ul_ref, il_ref, o_ref):
        m = m_ref[...]
        itf = it_ref[...]
        for j in range(_M):
            itf = itf + m[:, j * _F:(j + 1) * _F]
        net = jnp.sum(u_ref[...] * itf, axis=1, keepdims=True)
        iota = lax.broadcasted_iota(jnp.int32, (_BBLK, _LANES), 1)
        ub = jnp.sum(jnp.where(iota == ul_ref[...], ubr_ref[...], 0.0),
                     axis=1, keepdims=True)
        ib = jnp.sum(jnp.where(iota == il_ref[...], ibr_ref[...], 0.0),
                     axis=1, keepdims=True)
        o_ref[...] = net + ub + ib

    return pl.pallas_call(
        body,
        grid=(_B // _BBLK,),
        in_specs=[
            pl.BlockSpec((_BBLK, _F), lambda i: (i, 0)),
            pl.BlockSpec((_BBLK, _F), lambda i: (i, 0)),
            pl.BlockSpec((_BBLK, _M * _F), lambda i: (i, 0)),
            pl.BlockSpec((_BBLK, _LANES), lambda i: (i, 0)),
            pl.BlockSpec((_BBLK, _LANES), lambda i: (i, 0)),
            pl.BlockSpec((_BBLK, 1), lambda i: (i, 0)),
            pl.BlockSpec((_BBLK, 1), lambda i: (i, 0)),
        ],
        out_specs=pl.BlockSpec((_BBLK, 1), lambda i: (i, 0)),
        out_shape=jax.ShapeDtypeStruct((_B, 1), jnp.float32),
    )(u_rows, i_rows, m_rows, ub_rows, ib_rows, u_lane, i_lane)


def kernel(user, item, metadata, user_table, item_table, meta_table,
           user_bias, item_bias):
    user = user.astype(jnp.int32)
    item = item.astype(jnp.int32)
    m_flat = metadata.astype(jnp.int32).reshape(-1)
    ub16 = user_bias.reshape(-1, _LANES)
    ib16 = item_bias.reshape(-1, _LANES)
    u_hi = user // _LANES
    i_hi = item // _LANES
    u_lane = (user % _LANES).reshape(_B, 1)
    i_lane = (item % _LANES).reshape(_B, 1)
    u_rows, i_rows, m_rows, ub_rows, ib_rows = _sc_gather(
        user, item, m_flat, u_hi, i_hi,
        user_table, item_table, meta_table, ub16, ib16)
    return _tc_reduce(u_rows, i_rows, m_rows.reshape(_B, _M * _F),
                      ub_rows, ib_rows, u_lane, i_lane)


# speedup vs baseline: 1.0867x; 1.0867x over previous
import functools

import jax
import jax.numpy as jnp
from jax import lax
from jax.experimental import pallas as pl
from jax.experimental.pallas import tpu as pltpu
from jax.experimental.pallas import tpu_sc as plsc

_B = 16384
_F = 32
_M = 5
_NC = 2
_NS = 16
_NW = _NC * _NS
_BPW = _B // _NW
_MPW = _BPW * _M
_L = 128

_N_USERS = 1000000
_N_ITEMS = 1000000
_N_META = 100000
_BIAS_ROWS = 7816


_S_BIG = 1 << 18
_S_META = 1 << 15


def _repack(tT, s):
    w = 8192
    grid = s // w
    n = tT.shape[1]
    last = (n + w - 1) // w - 1
    specs = [pl.BlockSpec((_F, w), functools.partial(
        lambda q, i: (0, jnp.minimum(q * grid + i, last)), q))
        for q in range(4)]

    def body(t0_ref, t1_ref, t2_ref, t3_ref, o_ref):
        o_ref[...] = jnp.concatenate(
            [jnp.swapaxes(t_ref[...], 0, 1)
             for t_ref in (t0_ref, t1_ref, t2_ref, t3_ref)], axis=1)

    return pl.pallas_call(
        body,
        grid=(grid,),
        in_specs=specs,
        out_specs=pl.BlockSpec((w, 4 * _F), lambda i: (i, 0)),
        out_shape=jax.ShapeDtypeStruct((s, 4 * _F), jnp.float32),
    )(tT, tT, tT, tT)


def _sc_gather(u_idx, i_idx, m_idx, ub_idx, ib_idx, gu, gi, gm, bu, bi):
    mesh = plsc.VectorSubcoreMesh(core_axis_name="c", subcore_axis_name="s")
    out_type = (
        jax.ShapeDtypeStruct((_B, _L), jnp.float32),
        jax.ShapeDtypeStruct((_B, _L), jnp.float32),
        jax.ShapeDtypeStruct((_B * _M, _L), jnp.float32),
        jax.ShapeDtypeStruct((_B, _L), jnp.float32),
        jax.ShapeDtypeStruct((_B, _L), jnp.float32),
    )
    scratch = [
        pltpu.VMEM((_BPW,), jnp.int32),
        pltpu.VMEM((_BPW, _L), jnp.float32),
        pltpu.SemaphoreType.DMA,
    ]

    @functools.partial(pl.kernel, mesh=mesh, out_type=out_type,
                       scratch_types=scratch)
    def k(u_idx_h, i_idx_h, m_idx_h, ub_idx_h, ib_idx_h,
          gu_h, gi_h, gm_h, bu_h, bi_h,
          u_out, i_out, m_out, ub_out, ib_out,
          idx_v, row_v, sem):
        wid = lax.axis_index("s") * _NC + lax.axis_index("c")
        base = wid * _BPW

        def gather(idx_h, off, table_h, out_h):
            pltpu.sync_copy(idx_h.at[pl.ds(off, _BPW)], idx_v)
            pltpu.async_copy(table_h.at[idx_v], row_v, sem).wait()
            pltpu.sync_copy(row_v, out_h.at[pl.ds(off, _BPW)])

        gather(u_idx_h, base, gu_h, u_out)
        gather(i_idx_h, base, gi_h, i_out)
        gather(ub_idx_h, base, bu_h, ub_out)
        gather(ib_idx_h, base, bi_h, ib_out)
        mbase = wid * _MPW
        for h in range(_M):
            gather(m_idx_h, mbase + h * _BPW, gm_h, m_out)

    return k(u_idx, i_idx, m_idx, ub_idx, ib_idx, gu, gi, gm, bu, bi)


_BBLK = 1024


def _tc_reduce(u_rows, i_rows, m_rows, ub_rows, ib_rows,
               u_q, i_q, m_q, u_lane, i_lane):
    def body(u_ref, it_ref, m_ref, ubr_ref, ibr_ref,
             uq_ref, iq_ref, mq_ref, ul_ref, il_ref, o_ref):
        def pick(rows, qv):
            acc = jnp.where(qv == 0, rows[:, :_F], 0.0)
            for q in range(1, 4):
                acc = acc + jnp.where(qv == q,
                                      rows[:, q * _F:(q + 1) * _F], 0.0)
            return acc

        u32 = pick(u_ref[...], uq_ref[...])
        it32 = pick(it_ref[...], iq_ref[...])
        m32 = pick(m_ref[...], mq_ref[...]).reshape(_BBLK, _M, _F).sum(axis=1)
        net = jnp.sum(u32 * (it32 + m32), axis=1, keepdims=True)
        lanes = lax.broadcasted_iota(jnp.int32, (_BBLK, _L), 1)
        ub = jnp.sum(jnp.where(lanes == ul_ref[...], ubr_ref[...], 0.0),
                     axis=1, keepdims=True)
        ib = jnp.sum(jnp.where(lanes == il_ref[...], ibr_ref[...], 0.0),
                     axis=1, keepdims=True)
        o_ref[...] = net + ub + ib

    return pl.pallas_call(
        body,
        grid=(_B // _BBLK,),
        in_specs=[
            pl.BlockSpec((_BBLK, _L), lambda i: (i, 0)),
            pl.BlockSpec((_BBLK, _L), lambda i: (i, 0)),
            pl.BlockSpec((_BBLK * _M, _L), lambda i: (i, 0)),
            pl.BlockSpec((_BBLK, _L), lambda i: (i, 0)),
            pl.BlockSpec((_BBLK, _L), lambda i: (i, 0)),
            pl.BlockSpec((_BBLK, 1), lambda i: (i, 0)),
            pl.BlockSpec((_BBLK, 1), lambda i: (i, 0)),
            pl.BlockSpec((_BBLK * _M, 1), lambda i: (i, 0)),
            pl.BlockSpec((_BBLK, 1), lambda i: (i, 0)),
            pl.BlockSpec((_BBLK, 1), lambda i: (i, 0)),
        ],
        out_specs=pl.BlockSpec((_BBLK, 1), lambda i: (i, 0)),
        out_shape=jax.ShapeDtypeStruct((_B, 1), jnp.float32),
    )(u_rows, i_rows, m_rows, ub_rows, ib_rows,
      u_q, i_q, m_q, u_lane, i_lane)


def kernel(user, item, metadata, user_table, item_table, meta_table,
           user_bias, item_bias):
    user = user.astype(jnp.int32)
    item = item.astype(jnp.int32)
    m_flat = metadata.astype(jnp.int32).reshape(-1)

    gu = _repack(user_table.T, _S_BIG)
    gi = _repack(item_table.T, _S_BIG)
    gm = _repack(meta_table.T, _S_META)
    bu = jnp.pad(user_bias.reshape(-1), (0, _BIAS_ROWS * _L - _N_USERS))
    bu = bu.reshape(_BIAS_ROWS, _L)
    bi = jnp.pad(item_bias.reshape(-1), (0, _BIAS_ROWS * _L - _N_ITEMS))
    bi = bi.reshape(_BIAS_ROWS, _L)

    u_rows, i_rows, m_rows, ub_rows, ib_rows = _sc_gather(
        user & (_S_BIG - 1), item & (_S_BIG - 1), m_flat & (_S_META - 1),
        user // _L, item // _L,
        gu, gi, gm, bu, bi)

    return _tc_reduce(
        u_rows, i_rows, m_rows, ub_rows, ib_rows,
        (user >> 18).reshape(_B, 1), (item >> 18).reshape(_B, 1),
        (m_flat >> 15).reshape(_B * _M, 1),
        (user % _L).reshape(_B, 1), (item % _L).reshape(_B, 1))


# speedup vs baseline: 2.1288x; 1.9589x over previous
import functools

import jax
import jax.numpy as jnp
from jax import lax
from jax.experimental import pallas as pl
from jax.experimental.pallas import tpu as pltpu
from jax.experimental.pallas import tpu_sc as plsc

_B = 16384
_F = 32
_M = 5
_NC = 2
_NS = 16
_NW = _NC * _NS
_BPW = _B // _NW
_MPW = _BPW * _M
_L = 128

_N_USERS = 1000000
_N_ITEMS = 1000000
_N_META = 100000
_BIAS_ROWS = 7816


_S_BIG = 1 << 18
_S_META = 1 << 15


def _repack(tT, s):
    w = 16384
    grid = s // w
    n = tT.shape[1]
    last = (n + w - 1) // w - 1
    specs = [pl.BlockSpec((_F, w), functools.partial(
        lambda q, i: (0, jnp.minimum(q * grid + i, last)), q))
        for q in range(4)]

    def body(t0_ref, t1_ref, t2_ref, t3_ref, o_ref):
        stacked = jnp.concatenate(
            [t0_ref[...], t1_ref[...], t2_ref[...], t3_ref[...]], axis=0)
        o_ref[...] = jnp.swapaxes(stacked, 0, 1)

    return pl.pallas_call(
        body,
        grid=(grid,),
        in_specs=specs,
        out_specs=pl.BlockSpec((w, 4 * _F), lambda i: (i, 0)),
        out_shape=jax.ShapeDtypeStruct((s, 4 * _F), jnp.float32),
    )(tT, tT, tT, tT)


def _sc_gather(u_idx, i_idx, m_idx, ub_idx, ib_idx, gu, gi, gm, bu, bi):
    mesh = plsc.VectorSubcoreMesh(core_axis_name="c", subcore_axis_name="s")
    out_type = (
        jax.ShapeDtypeStruct((_B, _L), jnp.float32),
        jax.ShapeDtypeStruct((_B, _L), jnp.float32),
        jax.ShapeDtypeStruct((_B * _M, _L), jnp.float32),
        jax.ShapeDtypeStruct((_B, _L), jnp.float32),
        jax.ShapeDtypeStruct((_B, _L), jnp.float32),
    )
    scratch = [
        pltpu.VMEM((_BPW,), jnp.int32),
        pltpu.VMEM((_BPW, _L), jnp.float32),
        pltpu.SemaphoreType.DMA,
    ]

    @functools.partial(pl.kernel, mesh=mesh, out_type=out_type,
                       scratch_types=scratch)
    def k(u_idx_h, i_idx_h, m_idx_h, ub_idx_h, ib_idx_h,
          gu_h, gi_h, gm_h, bu_h, bi_h,
          u_out, i_out, m_out, ub_out, ib_out,
          idx_v, row_v, sem):
        wid = lax.axis_index("s") * _NC + lax.axis_index("c")
        base = wid * _BPW

        def gather(idx_h, off, table_h, out_h):
            pltpu.sync_copy(idx_h.at[pl.ds(off, _BPW)], idx_v)
            pltpu.async_copy(table_h.at[idx_v], row_v, sem).wait()
            pltpu.sync_copy(row_v, out_h.at[pl.ds(off, _BPW)])

        gather(u_idx_h, base, gu_h, u_out)
        gather(i_idx_h, base, gi_h, i_out)
        gather(ub_idx_h, base, bu_h, ub_out)
        gather(ib_idx_h, base, bi_h, ib_out)
        mbase = wid * _MPW
        for h in range(_M):
            gather(m_idx_h, mbase + h * _BPW, gm_h, m_out)

    return k(u_idx, i_idx, m_idx, ub_idx, ib_idx, gu, gi, gm, bu, bi)


_BBLK = 1024


def _tc_reduce(u_rows, i_rows, m_rows, ub_rows, ib_rows,
               u_q, i_q, m_q, u_lane, i_lane):
    def body(u_ref, it_ref, m_ref, ubr_ref, ibr_ref,
             uq_ref, iq_ref, mq_ref, ul_ref, il_ref, o_ref):
        grp = lax.broadcasted_iota(jnp.int32, (_BBLK, _L), 1) // _F
        u_m = jnp.where(grp == uq_ref[...], u_ref[...], 0.0)
        it_m = jnp.where(grp == iq_ref[...], it_ref[...], 0.0)
        grpm = lax.broadcasted_iota(jnp.int32, (_BBLK * _M, _L), 1) // _F
        m_sum = jnp.where(grpm == mq_ref[...], m_ref[...], 0.0)
        m_sum = m_sum.reshape(_BBLK, _M, _L).sum(axis=1)
        r = (lax.broadcasted_iota(jnp.int32, (_L, _L), 0) % _F
             == lax.broadcasted_iota(jnp.int32, (_L, _L), 1) % _F
             ).astype(jnp.float32)
        u_rep = lax.dot_general(u_m, r, (((1,), (0,)), ((), ())),
                                precision=lax.Precision.HIGHEST,
                                preferred_element_type=jnp.float32)
        net = jnp.sum(u_rep * (it_m + m_sum), axis=1, keepdims=True)
        lanes = lax.broadcasted_iota(jnp.int32, (_BBLK, _L), 1)
        ub = jnp.sum(jnp.where(lanes == ul_ref[...], ubr_ref[...], 0.0),
                     axis=1, keepdims=True)
        ib = jnp.sum(jnp.where(lanes == il_ref[...], ibr_ref[...], 0.0),
                     axis=1, keepdims=True)
        o_ref[...] = net + ub + ib

    return pl.pallas_call(
        body,
        grid=(_B // _BBLK,),
        in_specs=[
            pl.BlockSpec((_BBLK, _L), lambda i: (i, 0)),
            pl.BlockSpec((_BBLK, _L), lambda i: (i, 0)),
            pl.BlockSpec((_BBLK * _M, _L), lambda i: (i, 0)),
            pl.BlockSpec((_BBLK, _L), lambda i: (i, 0)),
            pl.BlockSpec((_BBLK, _L), lambda i: (i, 0)),
            pl.BlockSpec((_BBLK, 1), lambda i: (i, 0)),
            pl.BlockSpec((_BBLK, 1), lambda i: (i, 0)),
            pl.BlockSpec((_BBLK * _M, 1), lambda i: (i, 0)),
            pl.BlockSpec((_BBLK, 1), lambda i: (i, 0)),
            pl.BlockSpec((_BBLK, 1), lambda i: (i, 0)),
        ],
        out_specs=pl.BlockSpec((_BBLK, 1), lambda i: (i, 0)),
        out_shape=jax.ShapeDtypeStruct((_B, 1), jnp.float32),
    )(u_rows, i_rows, m_rows, ub_rows, ib_rows,
      u_q, i_q, m_q, u_lane, i_lane)


def kernel(user, item, metadata, user_table, item_table, meta_table,
           user_bias, item_bias):
    user = user.astype(jnp.int32)
    item = item.astype(jnp.int32)
    m_flat = metadata.astype(jnp.int32).reshape(-1)

    gu = _repack(user_table.T, _S_BIG)
    gi = _repack(item_table.T, _S_BIG)
    gm = _repack(meta_table.T, _S_META)
    bu = jnp.pad(user_bias.reshape(-1), (0, _BIAS_ROWS * _L - _N_USERS))
    bu = bu.reshape(_BIAS_ROWS, _L)
    bi = jnp.pad(item_bias.reshape(-1), (0, _BIAS_ROWS * _L - _N_ITEMS))
    bi = bi.reshape(_BIAS_ROWS, _L)

    u_rows, i_rows, m_rows, ub_rows, ib_rows = _sc_gather(
        user & (_S_BIG - 1), item & (_S_BIG - 1), m_flat & (_S_META - 1),
        user // _L, item // _L,
        gu, gi, gm, bu, bi)

    return _tc_reduce(
        u_rows, i_rows, m_rows, ub_rows, ib_rows,
        (user >> 18).reshape(_B, 1), (item >> 18).reshape(_B, 1),
        (m_flat >> 15).reshape(_B * _M, 1),
        (user % _L).reshape(_B, 1), (item % _L).reshape(_B, 1))
